# baseline (speedup 1.0000x reference)
import functools

import jax
import jax.numpy as jnp
from jax import lax
from jax.experimental import pallas as pl
from jax.experimental.pallas import tpu as pltpu
from jax.experimental.pallas import tpu_sc as plsc

N = 10000
N_PAD = 10240
ACC_N = 10112
D = 128
E = 320000
NC, NS, L = 2, 16, 16
NW = NC * NS
EPW = E // NW
CHUNK = 128
NFULL = 80
PH = 2
CPP = NFULL // PH
PPP = CPP // 2
RPT = N_PAD // NS
RPA = ACC_N // NS

_mesh = plsc.VectorSubcoreMesh(core_axis_name="c", subcore_axis_name="s")


@functools.partial(
    pl.kernel,
    out_type=jax.ShapeDtypeStruct((NC, N_PAD), jnp.float32),
    mesh=_mesh,
    scratch_types=[
        pltpu.VMEM((NFULL, CHUNK), jnp.int32),
        pltpu.VMEM((CHUNK,), jnp.float32),
        pltpu.VMEM((RPT,), jnp.float32),
        pltpu.SemaphoreType.DMA,
        pltpu.SemaphoreType.DMA,
        pltpu.VMEM_SHARED((N_PAD,), jnp.float32),
    ],
)
def _deg_call(dstp_hbm, out_hbm, didx2, ones_v, zbuf, isem, asem, deg_sp):
    c = lax.axis_index("c")
    s = lax.axis_index("s")
    wid = s * NC + c

    cp = pltpu.async_copy(dstp_hbm.at[wid], didx2, isem)

    zero16 = jnp.zeros((L,), jnp.float32)
    one16 = jnp.ones((L,), jnp.float32)
    for j in range(RPT // L):
        zbuf[pl.ds(j * L, L)] = zero16
    for j in range(CHUNK // L):
        ones_v[pl.ds(j * L, L)] = one16

    pltpu.sync_copy(zbuf, deg_sp.at[pl.ds(s * RPT, RPT)])
    cp.wait()
    plsc.subcore_barrier()

    def body(g, _):
        base = g * 8
        for k in range(8):
            pltpu.async_copy(ones_v, deg_sp.at[didx2.at[base + k]], asem,
                             add=True)
        for k in range(8):
            pltpu.make_async_copy(ones_v, deg_sp.at[didx2.at[0]], asem).wait()
        return ()

    lax.fori_loop(0, NFULL // 8, body, ())

    plsc.subcore_barrier()
    pltpu.sync_copy(deg_sp.at[pl.ds(s * RPT, RPT)],
                    out_hbm.at[c, pl.ds(s * RPT, RPT)])


@functools.partial(
    pl.kernel,
    out_type=jax.ShapeDtypeStruct((NC, N_PAD, D), jnp.float32),
    mesh=_mesh,
    scratch_types=[
        pltpu.VMEM((CPP, CHUNK), jnp.int32),
        pltpu.VMEM((CPP, CHUNK), jnp.int32),
        pltpu.VMEM((CHUNK, D), jnp.float32),
        pltpu.VMEM((CHUNK, D), jnp.float32),
        pltpu.SemaphoreType.DMA,
        pltpu.SemaphoreType.DMA,
        pltpu.SemaphoreType.DMA,
        pltpu.VMEM_SHARED((ACC_N, D), jnp.float32),
    ],
)
def _msg_call(g_hbm, srcp_hbm, dstp_hbm, out_hbm,
              sidx2, didx2, rows_a, rows_b, isem, ga, gb, acc_sp):
    c = lax.axis_index("c")
    s = lax.axis_index("s")
    wid = s * NC + c

    zero16 = jnp.zeros((L,), jnp.float32)

    def zbody(t, _):
        r = t // (D // L)
        k = t % (D // L)
        rows_a[r, pl.ds(k * L, L)] = zero16
        return ()

    lax.fori_loop(0, CHUNK * (D // L), zbody, ())
    zoff = 0
    for zlen in [CHUNK] * (RPA // CHUNK) + [RPA % CHUNK]:
        pltpu.sync_copy(rows_a.at[pl.ds(0, zlen)],
                        acc_sp.at[pl.ds(s * RPA + zoff, zlen)])
        zoff += zlen
    plsc.subcore_barrier()

    for h in range(PH):
        cp_s = pltpu.async_copy(srcp_hbm.at[wid, pl.ds(h * CPP, CPP)],
                                sidx2, isem)
        cp_d = pltpu.async_copy(dstp_hbm.at[wid, pl.ds(h * CPP, CPP)],
                                didx2, isem)
        cp_s.wait()
        cp_d.wait()

        def chunk_body(i, _):
            pltpu.async_copy(g_hbm.at[sidx2.at[i]], rows_a, ga).wait()
            pltpu.sync_copy(rows_a, acc_sp.at[didx2.at[i]], add=True)
            return ()

        lax.fori_loop(0, CPP, chunk_body, ())

    plsc.subcore_barrier()
    pltpu.sync_copy(acc_sp.at[pl.ds(s * RPA, RPA)],
                    out_hbm.at[c, pl.ds(s * RPA, RPA)])


BLK = 1024


def _mm_body(x_ref, w_ref, ds_ref, g_ref):
    dinv = lax.rsqrt(ds_ref[...] + 1.0)
    h = jnp.dot(x_ref[...], w_ref[...], preferred_element_type=jnp.float32)
    g_ref[...] = h * dinv


def _mm_call(x, W, dsum):
    return pl.pallas_call(
        _mm_body,
        grid=(N_PAD // BLK,),
        in_specs=[
            pl.BlockSpec((BLK, D), lambda i: (i, 0)),
            pl.BlockSpec((D, D), lambda i: (0, 0)),
            pl.BlockSpec((BLK, 1), lambda i: (i, 0)),
        ],
        out_specs=pl.BlockSpec((BLK, D), lambda i: (i, 0)),
        out_shape=jax.ShapeDtypeStruct((N_PAD, D), jnp.float32),
    )(x, W, dsum)


def _out_body(acc_ref, g_ref, ds_ref, b_ref, a_ref, o_ref):
    ssum = acc_ref[0] + acc_ref[1] + g_ref[...]
    dinv = lax.rsqrt(ds_ref[...] + 1.0)
    y = ssum * dinv + b_ref[...]
    o_ref[...] = jnp.where(y >= 0, y, a_ref[0, 0] * y)


def _out_call(accp, g, dsum, b2, a2):
    return pl.pallas_call(
        _out_body,
        grid=(N_PAD // BLK,),
        in_specs=[
            pl.BlockSpec((NC, BLK, D), lambda i: (0, i, 0)),
            pl.BlockSpec((BLK, D), lambda i: (i, 0)),
            pl.BlockSpec((BLK, 1), lambda i: (i, 0)),
            pl.BlockSpec((1, D), lambda i: (0, 0)),
            pl.BlockSpec((1, 1), lambda i: (0, 0)),
        ],
        out_specs=pl.BlockSpec((BLK, D), lambda i: (i, 0)),
        out_shape=jax.ShapeDtypeStruct((N_PAD, D), jnp.float32),
    )(accp, g, dsum, b2, a2)


def kernel(x, edge_index, W, b, a):
    src = edge_index[0].astype(jnp.int32)
    dst = edge_index[1].astype(jnp.int32)
    x_pad = jnp.zeros((N_PAD, D), x.dtype).at[:N].set(x)

    padv = jnp.full((NW, NFULL * CHUNK - EPW), N, jnp.int32)
    srcp = jnp.concatenate([src.reshape(NW, EPW), padv], 1)
    srcp = srcp.reshape(NW, NFULL, CHUNK)
    dstp = jnp.concatenate([dst.reshape(NW, EPW), padv], 1)
    dstp = dstp.reshape(NW, NFULL, CHUNK)

    degp = _deg_call(dstp)
    dsum = (degp[0] + degp[1])[:, None]
    g = _mm_call(x_pad, W, dsum)
    accp = _msg_call(g, srcp, dstp)
    out = _out_call(accp, g, dsum,
                    b.reshape(1, D).astype(jnp.float32),
                    a.reshape(1, 1).astype(jnp.float32))
    return out[:N]


# speedup vs baseline: 1.0888x; 1.0888x over previous
import functools

import jax
import jax.numpy as jnp
from jax import lax
from jax.experimental import pallas as pl
from jax.experimental.pallas import tpu as pltpu
from jax.experimental.pallas import tpu_sc as plsc

N = 10000
N_PAD = 10240
ACC_N = 10112
D = 128
E = 320000
NC, NS, L = 2, 16, 16
NW = NC * NS
EPW = E // NW
CHUNK = 128
NFULL = 80
PH = 2
CPP = NFULL // PH
PPP = CPP // 2
RPT = N_PAD // NS
RPA = ACC_N // NS

_mesh = plsc.VectorSubcoreMesh(core_axis_name="c", subcore_axis_name="s")


@functools.partial(
    pl.kernel,
    out_type=jax.ShapeDtypeStruct((NC, N_PAD), jnp.float32),
    mesh=_mesh,
    scratch_types=[
        pltpu.VMEM((NFULL, CHUNK), jnp.int32),
        pltpu.VMEM((CHUNK,), jnp.float32),
        pltpu.VMEM((RPT,), jnp.float32),
        pltpu.SemaphoreType.DMA,
        pltpu.SemaphoreType.DMA,
        pltpu.VMEM_SHARED((N_PAD,), jnp.float32),
    ],
)
def _deg_call(dstp_hbm, out_hbm, didx2, ones_v, zbuf, isem, asem, deg_sp):
    c = lax.axis_index("c")
    s = lax.axis_index("s")
    wid = s * NC + c

    cp = pltpu.async_copy(dstp_hbm.at[wid], didx2, isem)

    zero16 = jnp.zeros((L,), jnp.float32)
    one16 = jnp.ones((L,), jnp.float32)
    for j in range(RPT // L):
        zbuf[pl.ds(j * L, L)] = zero16
    for j in range(CHUNK // L):
        ones_v[pl.ds(j * L, L)] = one16

    pltpu.sync_copy(zbuf, deg_sp.at[pl.ds(s * RPT, RPT)])
    cp.wait()
    plsc.subcore_barrier()

    def body(g, _):
        base = g * 8
        for k in range(8):
            pltpu.async_copy(ones_v, deg_sp.at[didx2.at[base + k]], asem,
                             add=True)
        for k in range(8):
            pltpu.make_async_copy(ones_v, deg_sp.at[didx2.at[0]], asem).wait()
        return ()

    lax.fori_loop(0, NFULL // 8, body, ())

    plsc.subcore_barrier()
    pltpu.sync_copy(deg_sp.at[pl.ds(s * RPT, RPT)],
                    out_hbm.at[c, pl.ds(s * RPT, RPT)])


@functools.partial(
    pl.kernel,
    out_type=jax.ShapeDtypeStruct((NC, N_PAD, D), jnp.float32),
    mesh=_mesh,
    scratch_types=[
        pltpu.VMEM((CHUNK,), jnp.int32),
        pltpu.VMEM((CHUNK,), jnp.int32),
        pltpu.VMEM((CHUNK,), jnp.int32),
        pltpu.VMEM((CHUNK,), jnp.int32),
        pltpu.VMEM((CHUNK, D), jnp.float32),
        pltpu.VMEM((CHUNK, D), jnp.float32),
        pltpu.SemaphoreType.DMA,
        pltpu.SemaphoreType.DMA,
        pltpu.SemaphoreType.DMA,
        pltpu.SemaphoreType.DMA,
        pltpu.VMEM_SHARED((ACC_N, D), jnp.float32),
    ],
)
def _msg_call(g_hbm, srcp_hbm, dstp_hbm, out_hbm,
              sa, da, sb, db, rows_a, rows_b, isa, isb, ga, gb, acc_sp):
    c = lax.axis_index("c")
    s = lax.axis_index("s")
    wid = s * NC + c

    def load_idx(i, sbuf, dbuf, sem):
        ii = jnp.where(i >= NFULL, i - NFULL, i)
        pltpu.async_copy(srcp_hbm.at[wid, ii], sbuf, sem)
        pltpu.async_copy(dstp_hbm.at[wid, ii], dbuf, sem)

    def wait_idx(sbuf, dbuf, sem):
        pltpu.make_async_copy(srcp_hbm.at[wid, 0], sbuf, sem).wait()
        pltpu.make_async_copy(dstp_hbm.at[wid, 0], dbuf, sem).wait()

    zero16 = jnp.zeros((L,), jnp.float32)

    def zbody(t, _):
        r = t // (D // L)
        k = t % (D // L)
        rows_a[r, pl.ds(k * L, L)] = zero16
        return ()

    lax.fori_loop(0, CHUNK * (D // L), zbody, ())
    zoff = 0
    for zlen in [CHUNK] * (RPA // CHUNK) + [RPA % CHUNK]:
        pltpu.sync_copy(rows_a.at[pl.ds(0, zlen)],
                        acc_sp.at[pl.ds(s * RPA + zoff, zlen)])
        zoff += zlen
    plsc.subcore_barrier()

    load_idx(0, sa, da, isa)
    wait_idx(sa, da, isa)
    pltpu.async_copy(g_hbm.at[sa], rows_a, ga)
    load_idx(1, sb, db, isb)

    def pair(p, _):
        i0 = 2 * p
        wait_idx(sb, db, isb)
        pltpu.make_async_copy(g_hbm.at[sa], rows_a, ga).wait()
        pltpu.async_copy(g_hbm.at[sb], rows_b, gb)
        pltpu.sync_copy(rows_a, acc_sp.at[da], add=True)
        load_idx(i0 + 2, sa, da, isa)
        wait_idx(sa, da, isa)
        pltpu.make_async_copy(g_hbm.at[sb], rows_b, gb).wait()
        pltpu.async_copy(g_hbm.at[sa], rows_a, ga)
        pltpu.sync_copy(rows_b, acc_sp.at[db], add=True)
        load_idx(i0 + 3, sb, db, isb)
        return ()

    lax.fori_loop(0, NFULL // 2, pair, ())
    pltpu.make_async_copy(g_hbm.at[sa], rows_a, ga).wait()
    wait_idx(sb, db, isb)

    plsc.subcore_barrier()
    pltpu.sync_copy(acc_sp.at[pl.ds(s * RPA, RPA)],
                    out_hbm.at[c, pl.ds(s * RPA, RPA)])


BLK = 1024


def _mm_body(x_ref, w_ref, ds_ref, g_ref):
    dinv = lax.rsqrt(ds_ref[...] + 1.0)
    h = jnp.dot(x_ref[...], w_ref[...], preferred_element_type=jnp.float32)
    g_ref[...] = h * dinv


def _mm_call(x, W, dsum):
    return pl.pallas_call(
        _mm_body,
        grid=(N_PAD // BLK,),
        in_specs=[
            pl.BlockSpec((BLK, D), lambda i: (i, 0)),
            pl.BlockSpec((D, D), lambda i: (0, 0)),
            pl.BlockSpec((BLK, 1), lambda i: (i, 0)),
        ],
        out_specs=pl.BlockSpec((BLK, D), lambda i: (i, 0)),
        out_shape=jax.ShapeDtypeStruct((N_PAD, D), jnp.float32),
    )(x, W, dsum)


def _out_body(acc_ref, g_ref, ds_ref, b_ref, a_ref, o_ref):
    ssum = acc_ref[0] + acc_ref[1] + g_ref[...]
    dinv = lax.rsqrt(ds_ref[...] + 1.0)
    y = ssum * dinv + b_ref[...]
    o_ref[...] = jnp.where(y >= 0, y, a_ref[0, 0] * y)


def _out_call(accp, g, dsum, b2, a2):
    return pl.pallas_call(
        _out_body,
        grid=(N_PAD // BLK,),
        in_specs=[
            pl.BlockSpec((NC, BLK, D), lambda i: (0, i, 0)),
            pl.BlockSpec((BLK, D), lambda i: (i, 0)),
            pl.BlockSpec((BLK, 1), lambda i: (i, 0)),
            pl.BlockSpec((1, D), lambda i: (0, 0)),
            pl.BlockSpec((1, 1), lambda i: (0, 0)),
        ],
        out_specs=pl.BlockSpec((BLK, D), lambda i: (i, 0)),
        out_shape=jax.ShapeDtypeStruct((N_PAD, D), jnp.float32),
    )(accp, g, dsum, b2, a2)


def kernel(x, edge_index, W, b, a):
    src = edge_index[0].astype(jnp.int32)
    dst = edge_index[1].astype(jnp.int32)
    x_pad = jnp.zeros((N_PAD, D), x.dtype).at[:N].set(x)

    padv = jnp.full((NW, NFULL * CHUNK - EPW), N, jnp.int32)
    srcp = jnp.concatenate([src.reshape(NW, EPW), padv], 1)
    srcp = srcp.reshape(NW, NFULL, CHUNK)
    dstp = jnp.concatenate([dst.reshape(NW, EPW), padv], 1)
    dstp = dstp.reshape(NW, NFULL, CHUNK)

    degp = _deg_call(dstp)
    dsum = (degp[0] + degp[1])[:, None]
    g = _mm_call(x_pad, W, dsum)
    accp = _msg_call(g, srcp, dstp)
    out = _out_call(accp, g, dsum,
                    b.reshape(1, D).astype(jnp.float32),
                    a.reshape(1, 1).astype(jnp.float32))
    return out[:N]


# speedup vs baseline: 2.6122x; 2.3991x over previous
import functools

import jax
import jax.numpy as jnp
from jax import lax
from jax.experimental import pallas as pl
from jax.experimental.pallas import tpu as pltpu
from jax.experimental.pallas import tpu_sc as plsc

N = 10000
N_PAD = 10240
ACC_N = 10112
D = 128
E = 320000
NC, NS, L = 2, 16, 16
NW = NC * NS
EPW = E // NW
CHUNK = 128
NFULL = 80
PH = 2
CPP = NFULL // PH
PPP = CPP // 2
RPT = N_PAD // NS
RPA = ACC_N // NS

_mesh = plsc.VectorSubcoreMesh(core_axis_name="c", subcore_axis_name="s")


@functools.partial(
    pl.kernel,
    out_type=jax.ShapeDtypeStruct((NC, N_PAD), jnp.float32),
    mesh=_mesh,
    scratch_types=[
        pltpu.VMEM((NFULL, CHUNK), jnp.int32),
        pltpu.VMEM((CHUNK,), jnp.float32),
        pltpu.VMEM((RPT,), jnp.float32),
        pltpu.SemaphoreType.DMA,
        pltpu.SemaphoreType.DMA,
        pltpu.VMEM_SHARED((N_PAD,), jnp.float32),
    ],
)
def _deg_call(dstp_hbm, out_hbm, didx2, ones_v, zbuf, isem, asem, deg_sp):
    c = lax.axis_index("c")
    s = lax.axis_index("s")
    wid = s * NC + c

    cp = pltpu.async_copy(dstp_hbm.at[wid], didx2, isem)

    zero16 = jnp.zeros((L,), jnp.float32)
    one16 = jnp.ones((L,), jnp.float32)
    for j in range(RPT // L):
        zbuf[pl.ds(j * L, L)] = zero16
    for j in range(CHUNK // L):
        ones_v[pl.ds(j * L, L)] = one16

    pltpu.sync_copy(zbuf, deg_sp.at[pl.ds(s * RPT, RPT)])
    cp.wait()
    plsc.subcore_barrier()

    def body(g, _):
        base = g * 8
        for k in range(8):
            pltpu.async_copy(ones_v, deg_sp.at[didx2.at[base + k]], asem,
                             add=True)
        for k in range(8):
            pltpu.make_async_copy(ones_v, deg_sp.at[didx2.at[0]], asem).wait()
        return ()

    lax.fori_loop(0, NFULL // 8, body, ())

    plsc.subcore_barrier()
    pltpu.sync_copy(deg_sp.at[pl.ds(s * RPT, RPT)],
                    out_hbm.at[c, pl.ds(s * RPT, RPT)])


@functools.partial(
    pl.kernel,
    out_type=jax.ShapeDtypeStruct((NC, N_PAD, D), jnp.float32),
    mesh=_mesh,
    scratch_types=[
        pltpu.VMEM((CHUNK,), jnp.int32),
        pltpu.VMEM((CHUNK,), jnp.int32),
        pltpu.VMEM((CHUNK,), jnp.int32),
        pltpu.VMEM((CHUNK,), jnp.int32),
        pltpu.VMEM((CHUNK, D), jnp.float32),
        pltpu.VMEM((CHUNK, D), jnp.float32),
        pltpu.SemaphoreType.DMA,
        pltpu.SemaphoreType.DMA,
        pltpu.SemaphoreType.DMA,
        pltpu.SemaphoreType.DMA,
        pltpu.VMEM_SHARED((ACC_N, D), jnp.float32),
    ],
)
def _msg_call(g_hbm, srcp_hbm, dstp_hbm, out_hbm,
              sa, da, sb, db, rows_a, rows_b, isa, isb, ga, gb, acc_sp):
    c = lax.axis_index("c")
    s = lax.axis_index("s")
    wid = s * NC + c

    def load_idx(i, sbuf, dbuf, sem):
        ii = jnp.where(i >= NFULL, i - NFULL, i)
        pltpu.async_copy(srcp_hbm.at[wid, ii], sbuf, sem)
        pltpu.async_copy(dstp_hbm.at[wid, ii], dbuf, sem)

    def wait_idx(sbuf, dbuf, sem):
        pltpu.make_async_copy(srcp_hbm.at[wid, 0], sbuf, sem).wait()
        pltpu.make_async_copy(dstp_hbm.at[wid, 0], dbuf, sem).wait()

    zero16 = jnp.zeros((L,), jnp.float32)

    def zbody(t, _):
        r = t // (D // L)
        k = t % (D // L)
        rows_a[r, pl.ds(k * L, L)] = zero16
        return ()

    lax.fori_loop(0, CHUNK * (D // L), zbody, ())
    zoff = 0
    for zlen in [CHUNK] * (RPA // CHUNK) + [RPA % CHUNK]:
        pltpu.sync_copy(rows_a.at[pl.ds(0, zlen)],
                        acc_sp.at[pl.ds(s * RPA + zoff, zlen)])
        zoff += zlen
    plsc.subcore_barrier()

    load_idx(0, sa, da, isa)
    wait_idx(sa, da, isa)
    pltpu.async_copy(g_hbm.at[sa], rows_a, ga)
    load_idx(1, sb, db, isb)

    def pair(p, _):
        i0 = 2 * p
        wait_idx(sb, db, isb)
        pltpu.make_async_copy(g_hbm.at[sa], rows_a, ga).wait()
        pltpu.async_copy(g_hbm.at[sb], rows_b, gb)
        pltpu.sync_copy(rows_a, acc_sp.at[da], add=True)
        load_idx(i0 + 2, sa, da, isa)
        wait_idx(sa, da, isa)
        pltpu.make_async_copy(g_hbm.at[sb], rows_b, gb).wait()
        pltpu.async_copy(g_hbm.at[sa], rows_a, ga)
        pltpu.sync_copy(rows_b, acc_sp.at[db], add=True)
        load_idx(i0 + 3, sb, db, isb)
        return ()

    lax.fori_loop(0, NFULL // 2, pair, ())
    pltpu.make_async_copy(g_hbm.at[sa], rows_a, ga).wait()
    wait_idx(sb, db, isb)

    plsc.subcore_barrier()
    pltpu.sync_copy(acc_sp.at[pl.ds(s * RPA, RPA)],
                    out_hbm.at[c, pl.ds(s * RPA, RPA)])


BLK = 1024


def _mm_body(x_ref, w_ref, ds_ref, g_ref):
    dinv = lax.rsqrt(ds_ref[...] + 1.0)
    h = jnp.dot(x_ref[...], w_ref[...], preferred_element_type=jnp.float32)
    g_ref[...] = h * dinv


def _mm_call(x, W, dsum):
    return pl.pallas_call(
        _mm_body,
        grid=(N_PAD // BLK,),
        in_specs=[
            pl.BlockSpec((BLK, D), lambda i: (i, 0)),
            pl.BlockSpec((D, D), lambda i: (0, 0)),
            pl.BlockSpec((BLK, 1), lambda i: (i, 0)),
        ],
        out_specs=pl.BlockSpec((BLK, D), lambda i: (i, 0)),
        out_shape=jax.ShapeDtypeStruct((N_PAD, D), jnp.float32),
    )(x, W, dsum)


def _out_body(acc_ref, g_ref, ds_ref, b_ref, a_ref, o_ref):
    ssum = acc_ref[0] + acc_ref[1] + g_ref[...]
    dinv = lax.rsqrt(ds_ref[...] + 1.0)
    y = ssum * dinv + b_ref[...]
    o_ref[...] = jnp.where(y >= 0, y, a_ref[0, 0] * y)


def _out_call(accp, g, dsum, b2, a2):
    return pl.pallas_call(
        _out_body,
        grid=(N_PAD // BLK,),
        in_specs=[
            pl.BlockSpec((NC, BLK, D), lambda i: (0, i, 0)),
            pl.BlockSpec((BLK, D), lambda i: (i, 0)),
            pl.BlockSpec((BLK, 1), lambda i: (i, 0)),
            pl.BlockSpec((1, D), lambda i: (0, 0)),
            pl.BlockSpec((1, 1), lambda i: (0, 0)),
        ],
        out_specs=pl.BlockSpec((BLK, D), lambda i: (i, 0)),
        out_shape=jax.ShapeDtypeStruct((N_PAD, D), jnp.float32),
    )(accp, g, dsum, b2, a2)


def kernel(x, edge_index, W, b, a):
    src = edge_index[0].astype(jnp.int32)
    dst = edge_index[1].astype(jnp.int32)
    x_pad = jnp.zeros((N_PAD, D), x.dtype).at[:N].set(x)

    npad_e = NFULL * CHUNK - EPW
    padv = N + (jnp.arange(npad_e, dtype=jnp.int32) % (ACC_N - N))
    padv = jnp.broadcast_to(padv[None, :], (NW, npad_e))
    srcp = jnp.concatenate([src.reshape(NW, EPW), padv], 1)
    srcp = srcp.reshape(NW, NFULL, CHUNK)
    dstp = jnp.concatenate([dst.reshape(NW, EPW), padv], 1)
    dstp = dstp.reshape(NW, NFULL, CHUNK)

    degp = _deg_call(dstp)
    dsum = (degp[0] + degp[1])[:, None]
    g = _mm_call(x_pad, W, dsum)
    accp = _msg_call(g, srcp, dstp)
    out = _out_call(accp, g, dsum,
                    b.reshape(1, D).astype(jnp.float32),
                    a.reshape(1, 1).astype(jnp.float32))
    return out[:N]


# speedup vs baseline: 2.7187x; 1.0408x over previous
import functools

import jax
import jax.numpy as jnp
from jax import lax
from jax.experimental import pallas as pl
from jax.experimental.pallas import tpu as pltpu
from jax.experimental.pallas import tpu_sc as plsc

N = 10000
N_PAD = 10240
ACC_N = 10112
D = 128
E = 320000
NC, NS, L = 2, 16, 16
NW = NC * NS
EPW = E // NW
CHUNK = 128
NFULL = EPW // CHUNK
TAIL = EPW - NFULL * CHUNK
ER = E // CHUNK
AW = 24
ARW = 104
ATL = ER - AW * ARW
RPT = N_PAD // NS
RPA = ACC_N // NS

_mesh = plsc.VectorSubcoreMesh(core_axis_name="c", subcore_axis_name="s")


@functools.partial(
    pl.kernel,
    out_type=jax.ShapeDtypeStruct((NC, N_PAD), jnp.float32),
    mesh=_mesh,
    scratch_types=[
        pltpu.VMEM((ARW, CHUNK), jnp.int32),
        pltpu.VMEM((CHUNK,), jnp.int32),
        pltpu.VMEM((CHUNK,), jnp.float32),
        pltpu.VMEM((RPT,), jnp.float32),
        pltpu.SemaphoreType.DMA,
        pltpu.SemaphoreType.DMA,
        pltpu.VMEM_SHARED((N_PAD,), jnp.float32),
    ],
)
def _deg_call(dst3_hbm, dtail_hbm, out_hbm, didx2, tbuf, ones_v, zbuf,
              isem, asem, deg_sp):
    c = lax.axis_index("c")
    s = lax.axis_index("s")
    wid = s * NC + c

    zero16 = jnp.zeros((L,), jnp.float32)
    one16 = jnp.ones((L,), jnp.float32)
    for j in range(RPT // L):
        zbuf[pl.ds(j * L, L)] = zero16
    for j in range(CHUNK // L):
        ones_v[pl.ds(j * L, L)] = one16

    pltpu.sync_copy(zbuf, deg_sp.at[pl.ds(s * RPT, RPT)])

    @pl.when(wid < AW)
    def _():
        pltpu.sync_copy(dst3_hbm.at[pl.ds(wid * ARW, ARW)], didx2)

    plsc.subcore_barrier()

    @pl.when(wid < AW)
    def _():
        def body(g, _):
            base = g * 8
            for k in range(8):
                pltpu.async_copy(ones_v, deg_sp.at[didx2.at[base + k]], asem,
                                 add=True)
            for k in range(8):
                pltpu.make_async_copy(ones_v, deg_sp.at[didx2.at[0]],
                                      asem).wait()
            return ()

        lax.fori_loop(0, ARW // 8, body, ())

    @pl.when(wid == AW)
    def _():
        for k in range(ATL):
            pltpu.sync_copy(dtail_hbm.at[pl.ds(k * CHUNK, CHUNK)], tbuf)
            pltpu.sync_copy(ones_v, deg_sp.at[tbuf], add=True)

    plsc.subcore_barrier()
    pltpu.sync_copy(deg_sp.at[pl.ds(s * RPT, RPT)],
                    out_hbm.at[c, pl.ds(s * RPT, RPT)])


@functools.partial(
    pl.kernel,
    out_type=jax.ShapeDtypeStruct((NC, N_PAD, D), jnp.float32),
    mesh=_mesh,
    scratch_types=[
        pltpu.VMEM((CHUNK,), jnp.int32),
        pltpu.VMEM((CHUNK,), jnp.int32),
        pltpu.VMEM((CHUNK,), jnp.int32),
        pltpu.VMEM((CHUNK,), jnp.int32),
        pltpu.VMEM((TAIL,), jnp.int32),
        pltpu.VMEM((TAIL,), jnp.int32),
        pltpu.VMEM((CHUNK, D), jnp.float32),
        pltpu.VMEM((CHUNK, D), jnp.float32),
        pltpu.VMEM((TAIL, D), jnp.float32),
        pltpu.SemaphoreType.DMA,
        pltpu.SemaphoreType.DMA,
        pltpu.SemaphoreType.DMA,
        pltpu.SemaphoreType.DMA,
        pltpu.VMEM_SHARED((ACC_N, D), jnp.float32),
    ],
)
def _msg_call(g_hbm, src1_hbm, dst1_hbm, out_hbm,
              sa, da, sb, db, st, dt, rows_a, rows_b, rows_t,
              isa, isb, ga, gb, acc_sp):
    c = lax.axis_index("c")
    s = lax.axis_index("s")
    wid = s * NC + c
    ebase = wid * EPW

    def load_idx(i, sbuf, dbuf, sem):
        off = pl.multiple_of(
            ebase + jnp.where(i >= NFULL, 0, i) * CHUNK, 16)
        pltpu.async_copy(src1_hbm.at[pl.ds(off, CHUNK)], sbuf, sem)
        pltpu.async_copy(dst1_hbm.at[pl.ds(off, CHUNK)], dbuf, sem)

    def wait_idx(sbuf, dbuf, sem):
        pltpu.make_async_copy(src1_hbm.at[pl.ds(0, CHUNK)], sbuf,
                              sem).wait()
        pltpu.make_async_copy(dst1_hbm.at[pl.ds(0, CHUNK)], dbuf,
                              sem).wait()

    zero16 = jnp.zeros((L,), jnp.float32)

    def zbody(t, _):
        r = t // (D // L)
        k = t % (D // L)
        rows_a[r, pl.ds(k * L, L)] = zero16
        return ()

    lax.fori_loop(0, CHUNK * (D // L), zbody, ())
    zoff = 0
    for zlen in [CHUNK] * (RPA // CHUNK) + [RPA % CHUNK]:
        pltpu.sync_copy(rows_a.at[pl.ds(0, zlen)],
                        acc_sp.at[pl.ds(s * RPA + zoff, zlen)])
        zoff += zlen
    plsc.subcore_barrier()

    load_idx(0, sa, da, isa)
    wait_idx(sa, da, isa)
    pltpu.async_copy(g_hbm.at[sa], rows_a, ga)
    load_idx(1, sb, db, isb)

    def pair(p, _):
        i0 = 2 * p
        wait_idx(sb, db, isb)
        pltpu.make_async_copy(g_hbm.at[sa], rows_a, ga).wait()
        pltpu.async_copy(g_hbm.at[sb], rows_b, gb)
        pltpu.sync_copy(rows_a, acc_sp.at[da], add=True)
        load_idx(i0 + 2, sa, da, isa)
        wait_idx(sa, da, isa)
        pltpu.make_async_copy(g_hbm.at[sb], rows_b, gb).wait()
        pltpu.async_copy(g_hbm.at[sa], rows_a, ga)
        pltpu.sync_copy(rows_b, acc_sp.at[db], add=True)
        load_idx(i0 + 3, sb, db, isb)
        return ()

    lax.fori_loop(0, NFULL // 2, pair, ())
    pltpu.make_async_copy(g_hbm.at[sa], rows_a, ga).wait()
    wait_idx(sb, db, isb)

    toff = pl.multiple_of(ebase + NFULL * CHUNK, 16)
    pltpu.sync_copy(src1_hbm.at[pl.ds(toff, TAIL)], st)
    pltpu.sync_copy(dst1_hbm.at[pl.ds(toff, TAIL)], dt)
    pltpu.async_copy(g_hbm.at[st], rows_t, ga).wait()
    pltpu.sync_copy(rows_t, acc_sp.at[dt], add=True)

    plsc.subcore_barrier()
    pltpu.sync_copy(acc_sp.at[pl.ds(s * RPA, RPA)],
                    out_hbm.at[c, pl.ds(s * RPA, RPA)])


BLK = 1024


def _mm_body(x_ref, w_ref, ds_ref, g_ref):
    dinv = lax.rsqrt(ds_ref[...] + 1.0)
    h = jnp.dot(x_ref[...], w_ref[...], preferred_element_type=jnp.float32)
    g_ref[...] = h * dinv


def _mm_call(x, W, dsum):
    return pl.pallas_call(
        _mm_body,
        grid=(N_PAD // BLK,),
        in_specs=[
            pl.BlockSpec((BLK, D), lambda i: (i, 0)),
            pl.BlockSpec((D, D), lambda i: (0, 0)),
            pl.BlockSpec((BLK, 1), lambda i: (i, 0)),
        ],
        out_specs=pl.BlockSpec((BLK, D), lambda i: (i, 0)),
        out_shape=jax.ShapeDtypeStruct((N_PAD, D), jnp.float32),
    )(x, W, dsum)


def _out_body(acc_ref, g_ref, ds_ref, b_ref, a_ref, o_ref):
    ssum = acc_ref[0] + acc_ref[1] + g_ref[...]
    dinv = lax.rsqrt(ds_ref[...] + 1.0)
    y = ssum * dinv + b_ref[...]
    o_ref[...] = jnp.where(y >= 0, y, a_ref[0, 0] * y)


def _out_call(accp, g, dsum, b2, a2):
    return pl.pallas_call(
        _out_body,
        grid=(N_PAD // BLK,),
        in_specs=[
            pl.BlockSpec((NC, BLK, D), lambda i: (0, i, 0)),
            pl.BlockSpec((BLK, D), lambda i: (i, 0)),
            pl.BlockSpec((BLK, 1), lambda i: (i, 0)),
            pl.BlockSpec((1, D), lambda i: (0, 0)),
            pl.BlockSpec((1, 1), lambda i: (0, 0)),
        ],
        out_specs=pl.BlockSpec((BLK, D), lambda i: (i, 0)),
        out_shape=jax.ShapeDtypeStruct((N, D), jnp.float32),
    )(accp, g, dsum, b2, a2)


def kernel(x, edge_index, W, b, a):
    eidx = edge_index.astype(jnp.int32)
    src1 = eidx[0]
    dst1 = eidx[1]
    dst3 = dst1.reshape(ER, CHUNK)
    dtail = dst1[AW * ARW * CHUNK:] + 0

    degp = _deg_call(dst3, dtail)
    dsum = (degp[0] + degp[1])[:, None]
    g = _mm_call(x, W, dsum)
    accp = _msg_call(g, src1, dst1)
    return _out_call(accp, g, dsum,
                     b.reshape(1, D).astype(jnp.float32),
                     a.reshape(1, 1).astype(jnp.float32))


# speedup vs baseline: 2.8583x; 1.0513x over previous
import functools

import jax
import jax.numpy as jnp
from jax import lax
from jax.experimental import pallas as pl
from jax.experimental.pallas import tpu as pltpu
from jax.experimental.pallas import tpu_sc as plsc

N = 10000
N_PAD = 10240
ACC_N = 10112
D = 128
E = 320000
NC, NS, L = 2, 16, 16
NW = NC * NS
EPW = E // NW
CHUNK = 128
NFULL = EPW // CHUNK
TAIL = EPW - NFULL * CHUNK
ER = E // CHUNK
AW = 24
ARW = 104
ATL = ER - AW * ARW
RPT = N_PAD // NS
RPA = ACC_N // NS

_mesh = plsc.VectorSubcoreMesh(core_axis_name="c", subcore_axis_name="s")


@functools.partial(
    pl.kernel,
    out_type=jax.ShapeDtypeStruct((NC, N_PAD), jnp.float32),
    mesh=_mesh,
    scratch_types=[
        pltpu.VMEM((ARW, CHUNK), jnp.int32),
        pltpu.VMEM((CHUNK,), jnp.int32),
        pltpu.VMEM((CHUNK,), jnp.float32),
        pltpu.VMEM((RPT,), jnp.float32),
        pltpu.SemaphoreType.DMA,
        pltpu.SemaphoreType.DMA,
        pltpu.VMEM_SHARED((N_PAD,), jnp.float32),
    ],
)
def _deg_call(dst3_hbm, dtail_hbm, out_hbm, didx2, tbuf, ones_v, zbuf,
              isem, asem, deg_sp):
    c = lax.axis_index("c")
    s = lax.axis_index("s")
    wid = s * NC + c

    zero16 = jnp.zeros((L,), jnp.float32)
    one16 = jnp.ones((L,), jnp.float32)
    for j in range(RPT // L):
        zbuf[pl.ds(j * L, L)] = zero16
    for j in range(CHUNK // L):
        ones_v[pl.ds(j * L, L)] = one16

    pltpu.sync_copy(zbuf, deg_sp.at[pl.ds(s * RPT, RPT)])

    @pl.when(wid < AW)
    def _():
        pltpu.sync_copy(dst3_hbm.at[pl.ds(wid * ARW, ARW)], didx2)

    plsc.subcore_barrier()

    @pl.when(wid < AW)
    def _():
        def body(g, _):
            base = g * 8
            for k in range(8):
                pltpu.async_copy(ones_v, deg_sp.at[didx2.at[base + k]], asem,
                                 add=True)
            for k in range(8):
                pltpu.make_async_copy(ones_v, deg_sp.at[didx2.at[0]],
                                      asem).wait()
            return ()

        lax.fori_loop(0, ARW // 8, body, ())

    @pl.when(wid == AW)
    def _():
        for k in range(ATL):
            pltpu.sync_copy(dtail_hbm.at[pl.ds(k * CHUNK, CHUNK)], tbuf)
            pltpu.sync_copy(ones_v, deg_sp.at[tbuf], add=True)

    plsc.subcore_barrier()
    pltpu.sync_copy(deg_sp.at[pl.ds(s * RPT, RPT)],
                    out_hbm.at[c, pl.ds(s * RPT, RPT)])


@functools.partial(
    pl.kernel,
    out_type=jax.ShapeDtypeStruct((NC, N_PAD, D), jnp.float32),
    mesh=_mesh,
    scratch_types=[
        pltpu.VMEM((CHUNK,), jnp.int32),
        pltpu.VMEM((CHUNK,), jnp.int32),
        pltpu.VMEM((CHUNK,), jnp.int32),
        pltpu.VMEM((CHUNK,), jnp.int32),
        pltpu.VMEM((CHUNK,), jnp.int32),
        pltpu.VMEM((CHUNK,), jnp.int32),
        pltpu.VMEM((TAIL,), jnp.int32),
        pltpu.VMEM((TAIL,), jnp.int32),
        pltpu.VMEM((CHUNK, D), jnp.float32),
        pltpu.VMEM((CHUNK, D), jnp.float32),
        pltpu.VMEM((CHUNK, D), jnp.float32),
        pltpu.SemaphoreType.DMA,
        pltpu.SemaphoreType.DMA,
        pltpu.SemaphoreType.DMA,
        pltpu.SemaphoreType.DMA,
        pltpu.SemaphoreType.DMA,
        pltpu.SemaphoreType.DMA,
        pltpu.VMEM_SHARED((ACC_N, D), jnp.float32),
    ],
)
def _msg_call(g_hbm, src1_hbm, dst1_hbm, out_hbm,
              sa, da, sb, db, sc_, dc_, st, dt, rows_a, rows_b, rows_c,
              isa, isb, isc, ga, gb, gc, acc_sp):
    c = lax.axis_index("c")
    s = lax.axis_index("s")
    wid = s * NC + c
    ebase = wid * EPW

    def load_idx(i, sbuf, dbuf, sem):
        off = pl.multiple_of(
            ebase + jnp.where(i >= NFULL, 0, i) * CHUNK, 16)
        pltpu.async_copy(src1_hbm.at[pl.ds(off, CHUNK)], sbuf, sem)
        pltpu.async_copy(dst1_hbm.at[pl.ds(off, CHUNK)], dbuf, sem)

    def wait_idx(sbuf, dbuf, sem):
        pltpu.make_async_copy(src1_hbm.at[pl.ds(0, CHUNK)], sbuf,
                              sem).wait()
        pltpu.make_async_copy(dst1_hbm.at[pl.ds(0, CHUNK)], dbuf,
                              sem).wait()

    zero16 = jnp.zeros((L,), jnp.float32)

    def zbody(t, _):
        r = t // (D // L)
        k = t % (D // L)
        rows_a[r, pl.ds(k * L, L)] = zero16
        return ()

    lax.fori_loop(0, CHUNK * (D // L), zbody, ())
    zoff = 0
    for zlen in [CHUNK] * (RPA // CHUNK) + [RPA % CHUNK]:
        pltpu.sync_copy(rows_a.at[pl.ds(0, zlen)],
                        acc_sp.at[pl.ds(s * RPA + zoff, zlen)])
        zoff += zlen
    plsc.subcore_barrier()

    load_idx(0, sa, da, isa)
    load_idx(1, sb, db, isb)
    load_idx(2, sc_, dc_, isc)
    wait_idx(sa, da, isa)
    pltpu.async_copy(g_hbm.at[sa], rows_a, ga)
    wait_idx(sb, db, isb)
    pltpu.async_copy(g_hbm.at[sb], rows_b, gb)

    bufs = [(sa, da, rows_a, isa, ga),
            (sb, db, rows_b, isb, gb),
            (sc_, dc_, rows_c, isc, gc)]

    def triple(q, _):
        t0 = 3 * q
        for k in range(3):
            si, di, ri, isi, gi = bufs[k]
            s2, d2, r2, is2, g2 = bufs[(k + 2) % 3]
            wait_idx(s2, d2, is2)
            pltpu.async_copy(g_hbm.at[s2], r2, g2)
            pltpu.make_async_copy(g_hbm.at[si], ri, gi).wait()
            pltpu.sync_copy(ri, acc_sp.at[di], add=True)
            load_idx(t0 + k + 3, si, di, isi)
        return ()

    lax.fori_loop(0, NFULL // 3, triple, ())
    pltpu.make_async_copy(g_hbm.at[sa], rows_a, ga).wait()
    pltpu.make_async_copy(g_hbm.at[sb], rows_b, gb).wait()
    wait_idx(sc_, dc_, isc)

    toff = pl.multiple_of(ebase + NFULL * CHUNK, 16)
    pltpu.sync_copy(src1_hbm.at[pl.ds(toff, TAIL)], st)
    pltpu.sync_copy(dst1_hbm.at[pl.ds(toff, TAIL)], dt)
    pltpu.async_copy(g_hbm.at[st], rows_a.at[pl.ds(0, TAIL)], ga).wait()
    pltpu.sync_copy(rows_a.at[pl.ds(0, TAIL)], acc_sp.at[dt], add=True)

    plsc.subcore_barrier()
    pltpu.sync_copy(acc_sp.at[pl.ds(s * RPA, RPA)],
                    out_hbm.at[c, pl.ds(s * RPA, RPA)])


BLK = 1024


def _mm_body(x_ref, w_ref, ds_ref, g_ref):
    dinv = lax.rsqrt(ds_ref[...] + 1.0)
    h = jnp.dot(x_ref[...], w_ref[...], preferred_element_type=jnp.float32)
    g_ref[...] = h * dinv


def _mm_call(x, W, dsum):
    return pl.pallas_call(
        _mm_body,
        grid=(N_PAD // BLK,),
        in_specs=[
            pl.BlockSpec((BLK, D), lambda i: (i, 0)),
            pl.BlockSpec((D, D), lambda i: (0, 0)),
            pl.BlockSpec((BLK, 1), lambda i: (i, 0)),
        ],
        out_specs=pl.BlockSpec((BLK, D), lambda i: (i, 0)),
        out_shape=jax.ShapeDtypeStruct((N_PAD, D), jnp.float32),
    )(x, W, dsum)


def _out_body(acc_ref, g_ref, ds_ref, b_ref, a_ref, o_ref):
    ssum = acc_ref[0] + acc_ref[1] + g_ref[...]
    dinv = lax.rsqrt(ds_ref[...] + 1.0)
    y = ssum * dinv + b_ref[...]
    o_ref[...] = jnp.where(y >= 0, y, a_ref[0, 0] * y)


def _out_call(accp, g, dsum, b2, a2):
    return pl.pallas_call(
        _out_body,
        grid=(N_PAD // BLK,),
        in_specs=[
            pl.BlockSpec((NC, BLK, D), lambda i: (0, i, 0)),
            pl.BlockSpec((BLK, D), lambda i: (i, 0)),
            pl.BlockSpec((BLK, 1), lambda i: (i, 0)),
            pl.BlockSpec((1, D), lambda i: (0, 0)),
            pl.BlockSpec((1, 1), lambda i: (0, 0)),
        ],
        out_specs=pl.BlockSpec((BLK, D), lambda i: (i, 0)),
        out_shape=jax.ShapeDtypeStruct((N, D), jnp.float32),
    )(accp, g, dsum, b2, a2)


def kernel(x, edge_index, W, b, a):
    eidx = edge_index.astype(jnp.int32)
    src1 = eidx[0]
    dst1 = eidx[1]
    dst3 = dst1.reshape(ER, CHUNK)
    dtail = dst1[AW * ARW * CHUNK:] + 0

    degp = _deg_call(dst3, dtail)
    dsum = (degp[0] + degp[1])[:, None]
    g = _mm_call(x, W, dsum)
    accp = _msg_call(g, src1, dst1)
    return _out_call(accp, g, dsum,
                     b.reshape(1, D).astype(jnp.float32),
                     a.reshape(1, 1).astype(jnp.float32))
